# Initial kernel scaffold; baseline (speedup 1.0000x reference)
import jax
import jax.numpy as jnp
from jax.experimental import pallas as pl


def kernel(input_seq, embed, W_x, W_h, b_h, W_out, b_out, W_if, b_if, beta_read, beta_write, M0):
    raise NotImplementedError("write your pallas kernel here")


# speedup vs baseline: 13.4916x; 13.4916x over previous
import jax
import jax.numpy as jnp
import numpy as np
from jax.experimental import pallas as pl
from jax.experimental.pallas import tpu as pltpu
from jax.experimental.pallas import tpu_sc as plsc

VOCAB = 8192
EMBED = 512
HIDDEN = 1024
SLOTS = 512
DIM = 128
HEADS = 4
TOPK = 8
B = 32
T = 32
N = B * T

_GW = 128
_HALF = EMBED // 2

_bf16 = jnp.bfloat16
_f32 = jnp.float32


def _bdot(a, b):
    return jnp.dot(a.astype(_bf16), b.astype(_bf16), preferred_element_type=_f32)


def _bdg(a, b, dims):
    return jax.lax.dot_general(a.astype(_bf16), b.astype(_bf16), dims,
                               preferred_element_type=_f32)


def _gather_embed(embed2, idx2d):
    mesh = plsc.VectorSubcoreMesh(core_axis_name="c", subcore_axis_name="s")

    @pl.kernel(
        out_type=jax.ShapeDtypeStruct((2 * N, _HALF), jnp.float32),
        mesh=mesh,
    )
    def gk(x_hbm, i_hbm, o_hbm):
        def body(i_vmem, o_vmem):
            pltpu.sync_copy(x_hbm.at[i_vmem.at[0]], o_vmem)

        pltpu.emit_pipeline(
            body,
            grid=(2 * N // _GW,),
            in_specs=[pl.BlockSpec((1, _GW), lambda i: (i, 0))],
            out_specs=[pl.BlockSpec((_GW, _HALF), lambda i: (i, 0))],
            core_axis_name=("c", "s"),
            dimension_semantics=(pltpu.PARALLEL,),
        )(i_hbm, o_hbm)

    return gk(embed2, idx2d)


def _softmax_topk(sc):
    iota = jax.lax.broadcasted_iota(jnp.int32, sc.shape, sc.ndim - 1)
    work = sc
    for _ in range(TOPK - 1):
        m = jnp.max(work, axis=-1, keepdims=True)
        first = jnp.min(jnp.where(work == m, iota, SLOTS), axis=-1, keepdims=True)
        work = jnp.where(iota == first, -jnp.inf, work)
    thresh = jnp.max(work, axis=-1, keepdims=True)
    m1 = jnp.max(sc, axis=-1, keepdims=True)
    z = jnp.where(sc >= thresh, jnp.exp(sc - m1), 0.0)
    return z / jnp.sum(z, axis=-1, keepdims=True)


def _scan_kernel(eg_ref, wx_ref, wh_ref, bh_ref, wkv_ref, bkv_ref,
                 m0_ref, sca_ref, h_out_ref, mem_scr, h_scr, r_scr):
    c_r = sca_ref[0, 0]
    c_w = sca_ref[0, 1]
    mem_scr[:] = jnp.broadcast_to(m0_ref[:][None], (B, SLOTS, DIM))
    h_scr[:] = jnp.zeros((B, HIDDEN), jnp.float32)
    r_scr[:] = jnp.zeros((B, DIM), jnp.float32)

    def step(t, _):
        h = h_scr[:]
        r = r_scr[:]
        e = eg_ref[pl.ds(t * B, B), :]
        ctrl = jnp.concatenate([e, r], axis=-1)
        h_new = jnp.tanh(_bdot(ctrl, wx_ref[:]) + _bdot(h, wh_ref[:]) + bh_ref[:])
        h_out_ref[pl.ds(t * B, B), :] = h_new
        kv = _bdot(h_new, wkv_ref[:]) + bkv_ref[:]
        g = jax.nn.sigmoid(kv[:, 3 * HEADS * DIM:3 * HEADS * DIM + 2 * HEADS])

        mem = mem_scr[:]
        rk = kv[:, :HEADS * DIM].reshape(B, HEADS, DIM)
        wk = kv[:, HEADS * DIM:2 * HEADS * DIM].reshape(B, HEADS, DIM)
        wv = jnp.tanh(kv[:, 2 * HEADS * DIM:3 * HEADS * DIM]).reshape(B, HEADS, DIM)
        sc = _bdg(rk, mem, (((2,), (2,)), ((0,), (0,)))) * c_r
        rw = _softmax_topk(sc)
        r_h = _bdg(rw, mem, (((2,), (1,)), ((0,), (0,))))
        read_new = (((r_h[:, 0] + r_h[:, 1]) + r_h[:, 2]) + r_h[:, 3]) * np.float32(0.25)
        scw = _bdg(wk, mem, (((2,), (2,)), ((0,), (0,)))) * c_w
        mw = jnp.max(scw, axis=-1, keepdims=True)
        zw = jnp.exp(scw - mw)
        ww = zw / jnp.sum(zw, axis=-1, keepdims=True)
        erase = g[:, :HEADS]
        addg = g[:, HEADS:]
        keep = 1.0 - ww[:, 0] * erase[:, 0:1]
        for hh in range(1, HEADS):
            keep = keep * (1.0 - ww[:, hh] * erase[:, hh:hh + 1])
        wwag = ww * addg[:, :, None]
        add = _bdg(wwag, wv, (((1,), (1,)), ((0,), (0,))))
        mem_scr[:] = mem * keep[:, :, None] + add
        r_scr[:] = read_new
        h_scr[:] = h_new
        return 0

    jax.lax.fori_loop(0, T, step, 0)


def _logits_kernel(h_ref, w_ref, b_ref, o_ref):
    o_ref[:] = _bdot(h_ref[:], w_ref[:]) + b_ref[:]


def kernel(input_seq, embed, W_x, W_h, b_h, W_out, b_out, W_if, b_if,
           beta_read, beta_write, M0):
    Wif3 = W_if.reshape(HIDDEN, HEADS, 3 * DIM + 2)
    bif3 = b_if.reshape(HEADS, 3 * DIM + 2)
    W_kv = jnp.concatenate(
        [
            Wif3[:, :, :DIM].reshape(HIDDEN, HEADS * DIM),
            Wif3[:, :, DIM:2 * DIM].reshape(HIDDEN, HEADS * DIM),
            Wif3[:, :, 2 * DIM:3 * DIM].reshape(HIDDEN, HEADS * DIM),
            Wif3[:, :, 3 * DIM],
            Wif3[:, :, 3 * DIM + 1],
        ],
        axis=1,
    )
    b_kv = jnp.concatenate(
        [
            bif3[:, :DIM].reshape(HEADS * DIM),
            bif3[:, DIM:2 * DIM].reshape(HEADS * DIM),
            bif3[:, 2 * DIM:3 * DIM].reshape(HEADS * DIM),
            bif3[:, 3 * DIM],
            bif3[:, 3 * DIM + 1],
        ]
    )[None, :]
    isd = np.float32(1.0 / np.sqrt(DIM))
    inf32 = jnp.float32(jnp.inf)
    scales = jnp.stack(
        [jnp.nextafter(jnp.logaddexp(beta_read, 0.0), inf32) * isd,
         jnp.nextafter(jnp.logaddexp(beta_write, 0.0), inf32) * isd]
    ).reshape(1, 2)

    idx = input_seq.T.reshape(-1).astype(jnp.int32)
    idx2d = (idx[:, None] * 2 + jnp.arange(2, dtype=jnp.int32)[None, :]).reshape(
        2 * N // _GW, _GW
    )
    Eg = _gather_embed(embed.reshape(2 * VOCAB, _HALF), idx2d).reshape(N, EMBED)

    Hmat = pl.pallas_call(
        _scan_kernel,
        out_shape=jax.ShapeDtypeStruct((N, HIDDEN), jnp.float32),
        scratch_shapes=[
            pltpu.VMEM((B, SLOTS, DIM), jnp.float32),
            pltpu.VMEM((B, HIDDEN), jnp.float32),
            pltpu.VMEM((B, DIM), jnp.float32),
        ],
    )(Eg, W_x, W_h, b_h[None, :], W_kv, b_kv, M0, scales)

    VB = 8
    VBLK = VOCAB // VB
    logits = pl.pallas_call(
        _logits_kernel,
        grid=(VB,),
        in_specs=[
            pl.BlockSpec((N, HIDDEN), lambda i: (0, 0)),
            pl.BlockSpec((HIDDEN, VBLK), lambda i: (0, i)),
            pl.BlockSpec((1, VBLK), lambda i: (0, i)),
        ],
        out_specs=pl.BlockSpec((N, VBLK), lambda i: (0, i)),
        out_shape=jax.ShapeDtypeStruct((N, VOCAB), jnp.float32),
    )(Hmat, W_out, b_out[None, :])

    return logits.reshape(T, B, VOCAB).transpose(1, 0, 2)
